# baseline (speedup 1.0000x reference)
import functools

import jax
import jax.numpy as jnp
from jax import lax
from jax.experimental import pallas as pl
from jax.experimental.pallas import tpu as pltpu
from jax.experimental.pallas import tpu_sc as plsc

B = 16384
H = 64
V = 100

_info = plsc.get_sparse_core_info()
_NC, _NS, _L = _info.num_cores, _info.num_subcores, _info.num_lanes
_NW = _NC * _NS
_BPW = B // _NW
_CH = 128
_NCHUNK = _BPW // _CH
_HC = H // _L


def _sc_kernel(z_hbm, z1_hbm, z2_hbm, z3_hbm, table_hbm,
               w1_hbm, b1_hbm, w2_hbm, b2_hbm, w3_hbm, b3_hbm,
               o0_hbm, o1_hbm, o2_hbm, o3_hbm,
               idx_v, rows_v, z1_v, z2_v, z3_v,
               out1_v, out2_v, out3_v, wb_v, sem):
    wid = lax.axis_index("s") * _NC + lax.axis_index("c")

    pltpu.sync_copy(w1_hbm, wb_v.at[0])
    pltpu.sync_copy(b1_hbm, wb_v.at[1])
    pltpu.sync_copy(w2_hbm, wb_v.at[2])
    pltpu.sync_copy(b2_hbm, wb_v.at[3])
    pltpu.sync_copy(w3_hbm, wb_v.at[4])
    pltpu.sync_copy(b3_hbm, wb_v.at[5])

    wr = [[wb_v[2 * i, pl.ds(c * _L, _L)] for c in range(_HC)]
          for i in range(3)]
    br = [[wb_v[2 * i + 1, pl.ds(c * _L, _L)] for c in range(_HC)]
          for i in range(3)]

    for k in range(_NCHUNK):
        base = wid * _BPW + k * _CH

        pltpu.sync_copy(z_hbm.at[pl.ds(base, _CH)], idx_v)
        gather = pltpu.async_copy(table_hbm.at[idx_v], rows_v, sem)

        pltpu.sync_copy(z1_hbm.at[pl.ds(base, _CH)], z1_v)
        pltpu.sync_copy(z2_hbm.at[pl.ds(base, _CH)], z2_v)
        pltpu.sync_copy(z3_hbm.at[pl.ds(base, _CH)], z3_v)

        def body(jj, carry, outs=(out1_v, out2_v, out3_v),
                 zs=(z1_v, z2_v, z3_v)):
            zvs = [z[pl.ds(jj * _L, _L)] for z in zs]
            for lane in range(_L):
                r = jj * _L + lane
                sel = jnp.full((_L,), lane, dtype=jnp.int32)
                for i in range(3):
                    zb = zvs[i].at[sel].get(mode="promise_in_bounds")
                    for c in range(_HC):
                        outs[i][r, pl.ds(c * _L, _L)] = zb * wr[i][c] + br[i][c]
            return carry

        lax.fori_loop(0, _CH // _L, body, 0)

        gather.wait()
        pltpu.sync_copy(rows_v, o0_hbm.at[pl.ds(base, _CH)])
        pltpu.sync_copy(out1_v, o1_hbm.at[pl.ds(base, _CH)])
        pltpu.sync_copy(out2_v, o2_hbm.at[pl.ds(base, _CH)])
        pltpu.sync_copy(out3_v, o3_hbm.at[pl.ds(base, _CH)])


def kernel(z, z1, z2, z3, emb_table, W1, b1, W2, b2, W3, b3):
    mesh = plsc.VectorSubcoreMesh(core_axis_name="c", subcore_axis_name="s")
    f32 = jnp.float32
    run = pl.kernel(
        _sc_kernel, mesh=mesh,
        out_type=(
            jax.ShapeDtypeStruct((B, H), f32),
            jax.ShapeDtypeStruct((B, H), f32),
            jax.ShapeDtypeStruct((B, H), f32),
            jax.ShapeDtypeStruct((B, H), f32),
        ),
        scratch_types=[
            pltpu.VMEM((_CH,), jnp.int32),
            pltpu.VMEM((_CH, H), f32),
            pltpu.VMEM((_CH,), f32),
            pltpu.VMEM((_CH,), f32),
            pltpu.VMEM((_CH,), f32),
            pltpu.VMEM((_CH, H), f32),
            pltpu.VMEM((_CH, H), f32),
            pltpu.VMEM((_CH, H), f32),
            pltpu.VMEM((6, H), f32),
            pltpu.SemaphoreType.DMA,
        ],
        compiler_params=pltpu.CompilerParams(use_tc_tiling_on_sc=False),
    )
    return run(z.astype(jnp.int32), z1.reshape(-1), z2.reshape(-1),
               z3.reshape(-1), emb_table,
               W1.reshape(-1), b1, W2.reshape(-1), b2, W3.reshape(-1), b3)


# speedup vs baseline: 1.3034x; 1.3034x over previous
import jax
import jax.numpy as jnp
from jax import lax
from jax.experimental import pallas as pl
from jax.experimental.pallas import tpu as pltpu
from jax.experimental.pallas import tpu_sc as plsc

B = 16384
H = 64
V = 100

_info = plsc.get_sparse_core_info()
_NC, _NS, _L = _info.num_cores, _info.num_subcores, _info.num_lanes
_NW = _NC * _NS
_BPW = B // _NW
_CH = 128
_NCHUNK = _BPW // _CH
_HC = H // _L


def _sc_kernel(z_hbm, z1_hbm, z2_hbm, z3_hbm, table_hbm,
               w1_hbm, b1_hbm, w2_hbm, b2_hbm, w3_hbm, b3_hbm,
               o0_hbm, o1_hbm, o2_hbm, o3_hbm,
               table_v, idx_v, z1_v, z2_v, z3_v,
               o0_v, o1_v, o2_v, o3_v, wb_v, sem):
    wid = lax.axis_index("s") * _NC + lax.axis_index("c")
    base_w = wid * _BPW

    pltpu.sync_copy(table_hbm, table_v)
    pltpu.sync_copy(z_hbm.at[pl.ds(base_w, _BPW)], idx_v)
    pltpu.sync_copy(z1_hbm.at[pl.ds(base_w, _BPW)], z1_v)
    pltpu.sync_copy(z2_hbm.at[pl.ds(base_w, _BPW)], z2_v)
    pltpu.sync_copy(z3_hbm.at[pl.ds(base_w, _BPW)], z3_v)
    pltpu.sync_copy(w1_hbm, wb_v.at[0])
    pltpu.sync_copy(b1_hbm, wb_v.at[1])
    pltpu.sync_copy(w2_hbm, wb_v.at[2])
    pltpu.sync_copy(b2_hbm, wb_v.at[3])
    pltpu.sync_copy(w3_hbm, wb_v.at[4])
    pltpu.sync_copy(b3_hbm, wb_v.at[5])

    wr = [[wb_v[2 * i, pl.ds(c * _L, _L)] for c in range(_HC)]
          for i in range(3)]
    br = [[wb_v[2 * i + 1, pl.ds(c * _L, _L)] for c in range(_HC)]
          for i in range(3)]
    cols = [lax.iota(jnp.int32, _L) + c * _L for c in range(_HC)]

    outs = (o1_v, o2_v, o3_v)
    zs = (z1_v, z2_v, z3_v)
    copies = []
    for k in range(_NCHUNK):
        for d in copies:
            d.wait()
        copies = []

        def body(jj, carry, k=k):
            row0 = k * _CH + jj * _L
            zi64 = idx_v[pl.ds(row0, _L)] * H
            zvs = [zref[pl.ds(row0, _L)] for zref in zs]
            for lane in range(_L):
                r = jj * _L + lane
                sel = jnp.full((_L,), lane, dtype=jnp.int32)
                rs = zi64.at[sel].get(mode="promise_in_bounds")
                for c in range(_HC):
                    g = plsc.load_gather(table_v, [rs + cols[c]])
                    o0_v[r, pl.ds(c * _L, _L)] = g
                for i in range(3):
                    zb = zvs[i].at[sel].get(mode="promise_in_bounds")
                    for c in range(_HC):
                        outs[i][r, pl.ds(c * _L, _L)] = zb * wr[i][c] + br[i][c]
            return carry

        lax.fori_loop(0, _CH // _L, body, 0)

        base = base_w + k * _CH
        copies = [
            pltpu.async_copy(o0_v, o0_hbm.at[pl.ds(base, _CH)], sem),
            pltpu.async_copy(o1_v, o1_hbm.at[pl.ds(base, _CH)], sem),
            pltpu.async_copy(o2_v, o2_hbm.at[pl.ds(base, _CH)], sem),
            pltpu.async_copy(o3_v, o3_hbm.at[pl.ds(base, _CH)], sem),
        ]
    for d in copies:
        d.wait()


def kernel(z, z1, z2, z3, emb_table, W1, b1, W2, b2, W3, b3):
    mesh = plsc.VectorSubcoreMesh(core_axis_name="c", subcore_axis_name="s")
    f32 = jnp.float32
    run = pl.kernel(
        _sc_kernel, mesh=mesh,
        out_type=(
            jax.ShapeDtypeStruct((B, H), f32),
            jax.ShapeDtypeStruct((B, H), f32),
            jax.ShapeDtypeStruct((B, H), f32),
            jax.ShapeDtypeStruct((B, H), f32),
        ),
        scratch_types=[
            pltpu.VMEM((V * H,), f32),
            pltpu.VMEM((_BPW,), jnp.int32),
            pltpu.VMEM((_BPW,), f32),
            pltpu.VMEM((_BPW,), f32),
            pltpu.VMEM((_BPW,), f32),
            pltpu.VMEM((_CH, H), f32),
            pltpu.VMEM((_CH, H), f32),
            pltpu.VMEM((_CH, H), f32),
            pltpu.VMEM((_CH, H), f32),
            pltpu.VMEM((6, H), f32),
            pltpu.SemaphoreType.DMA,
        ],
        compiler_params=pltpu.CompilerParams(needs_layout_passes=False),
    )
    return run(z.astype(jnp.int32), z1.reshape(-1), z2.reshape(-1),
               z3.reshape(-1), emb_table.reshape(-1),
               W1.reshape(-1), b1, W2.reshape(-1), b2, W3.reshape(-1), b3)
